# baseline (speedup 1.0000x reference)
import functools

import jax
import jax.numpy as jnp
from jax import lax
from jax.experimental import pallas as pl
from jax.experimental.pallas import tpu as pltpu
from jax.experimental.pallas import tpu_sc as plsc

N = 10000
E = 320000
IN_DIM = 128
HID_DIM = 64
OUT_DIM = 8

NC = 2
NS = 16
NW = NC * NS
CHUNK = 128
NCHUNK = 80
EPW = NCHUNK * CHUNK
E_PAD = NW * EPW
NBUF = 5
N_PAD = 10240
ROWS_PER_TILE = N_PAD // NS
ZCHUNK = 128
PAD_ROW = N_PAD - 1

D1 = 72
D2 = 16


def _edge_pass(dp: int):
    mesh = plsc.VectorSubcoreMesh(core_axis_name="c", subcore_axis_name="s")

    @functools.partial(
        pl.kernel,
        out_type=jax.ShapeDtypeStruct((NC, N_PAD, 128), jnp.float32),
        mesh=mesh,
        scratch_types=[
            pltpu.VMEM((NCHUNK, CHUNK), jnp.int32),
            pltpu.VMEM((NCHUNK, CHUNK), jnp.int32),
            pltpu.VMEM((NBUF, CHUNK, dp), jnp.float32),
            pltpu.VMEM((ZCHUNK, dp), jnp.float32),
            pltpu.VMEM_SHARED((N_PAD, dp), jnp.float32),
        ] + [pltpu.SemaphoreType.DMA] * (2 * NBUF),
        compiler_params=pltpu.CompilerParams(use_tc_tiling_on_sc=False),
    )
    def k(y_hbm, ei_hbm, part_hbm, idx_s, idx_d, rows, zbuf, acc,
          *sems):
        c = lax.axis_index("c")
        s = lax.axis_index("s")
        wid = c * NS + s

        pltpu.sync_copy(ei_hbm.at[0, wid], idx_s)
        pltpu.sync_copy(ei_hbm.at[1, wid], idx_d)

        zoffs = list(range(0, dp - 15, 16))
        if zoffs[-1] + 16 < dp:
            zoffs.append(dp - 16)

        @pl.loop(0, ZCHUNK)
        def _zrows(i):
            for j in zoffs:
                zbuf[i, pl.ds(j, 16)] = jnp.zeros((16,), jnp.float32)

        @pl.loop(0, ROWS_PER_TILE // ZCHUNK)
        def _zacc(kk):
            pltpu.sync_copy(zbuf, acc.at[pl.ds(s * ROWS_PER_TILE + kk * ZCHUNK, ZCHUNK)])

        plsc.subcore_barrier()

        gsems = sems[:NBUF]
        ssems = sems[NBUF:]
        LOOK = 3
        for b in range(LOOK):
            pltpu.async_copy(y_hbm.at[idx_s.at[b]], rows.at[b], gsems[b])

        @pl.loop(0, NCHUNK // NBUF)
        def _p(p):
            for b in range(NBUF):
                g = p * NBUF + b
                bf = (b + LOOK) % NBUF
                gf = g + LOOK

                @pl.when(g >= NBUF - LOOK)
                def _drain():
                    pltpu.make_async_copy(rows.at[bf], acc.at[idx_d.at[0]],
                                          ssems[bf]).wait()

                @pl.when(gf < NCHUNK)
                def _refill():
                    pltpu.async_copy(y_hbm.at[idx_s.at[gf]], rows.at[bf],
                                     gsems[bf])

                pltpu.make_async_copy(y_hbm.at[idx_s.at[g]], rows.at[b],
                                      gsems[b]).wait()
                pltpu.async_copy(rows.at[b], acc.at[idx_d.at[g]], ssems[b],
                                 add=True)

        for q in range(NCHUNK - NBUF + LOOK, NCHUNK):
            pltpu.make_async_copy(rows.at[q % NBUF], acc.at[idx_d.at[0]],
                                  ssems[q % NBUF]).wait()

        plsc.subcore_barrier()
        pltpu.sync_copy(
            acc.at[pl.ds(s * ROWS_PER_TILE, ROWS_PER_TILE)],
            part_hbm.at[c, pl.ds(s * ROWS_PER_TILE, ROWS_PER_TILE),
                        pl.ds(0, dp)],
        )

    return k


_edge_pass_1 = _edge_pass(D1)
_edge_pass_2 = _edge_pass(D2)


def _tc_a_body(x_ref, wl_ref, wr_ref, b1_ref, y1_ref, xw1r_ref):
    xv = x_ref[...]
    y = jnp.dot(xv, wl_ref[...], preferred_element_type=jnp.float32)
    col = lax.broadcasted_iota(jnp.int32, (N, D1 - HID_DIM), 1)
    pad = jnp.where(col == 0, 1.0, 0.0).astype(jnp.float32)
    y1_ref[...] = jnp.concatenate([y, pad], axis=1)
    xw1r_ref[...] = (jnp.dot(xv, wr_ref[...],
                             preferred_element_type=jnp.float32) + b1_ref[...])


_tc_a = pl.pallas_call(
    _tc_a_body,
    out_shape=(
        jax.ShapeDtypeStruct((N, D1), jnp.float32),
        jax.ShapeDtypeStruct((N, HID_DIM), jnp.float32),
    ),
)


def _tc_b_body(part_ref, xw1r_ref, w2l_ref, w2r_ref, b2_ref,
               y2_ref, z_ref, dinv_ref):
    p = part_ref[0, :N] + part_ref[1, :N]
    deg = p[:, HID_DIM:HID_DIM + 1]
    dinv = 1.0 / jnp.maximum(deg, 1.0)
    h = jnp.maximum(p[:, :HID_DIM] * dinv + xw1r_ref[...], 0.0)
    y2 = jnp.dot(h, w2l_ref[...], preferred_element_type=jnp.float32)
    y2_ref[...] = jnp.concatenate(
        [y2, jnp.zeros((N, D2 - OUT_DIM), jnp.float32)], axis=1)
    z_ref[...] = (jnp.dot(h, w2r_ref[...],
                          preferred_element_type=jnp.float32) + b2_ref[...])
    dinv_ref[...] = jnp.broadcast_to(dinv, (N, OUT_DIM))


_tc_b = pl.pallas_call(
    _tc_b_body,
    out_shape=(
        jax.ShapeDtypeStruct((N, D2), jnp.float32),
        jax.ShapeDtypeStruct((N, OUT_DIM), jnp.float32),
        jax.ShapeDtypeStruct((N, OUT_DIM), jnp.float32),
    ),
)


def _tc_c_body(part_ref, dinv_ref, z_ref, o_ref):
    p = part_ref[0, :N] + part_ref[1, :N]
    o_ref[...] = p[:, :OUT_DIM] * dinv_ref[...] + z_ref[...]


_tc_c = pl.pallas_call(
    _tc_c_body,
    out_shape=jax.ShapeDtypeStruct((N, OUT_DIM), jnp.float32),
)


def kernel(x, edge_index, W1_l, W1_r, b1, W2_l, W2_r, b2):
    npad = E_PAD - E
    pad_iota = jax.lax.iota(jnp.int32, npad)
    pad_dst = N + pad_iota % (N_PAD - N)
    pad_src = pad_iota % N
    pad_block = jnp.stack([pad_src, pad_dst])
    ei = jnp.concatenate([edge_index, pad_block], axis=1).reshape(
        2, NW, NCHUNK, CHUNK)
    y1, xw1r = _tc_a(x, W1_l, W1_r, b1.reshape(1, HID_DIM))
    part1 = _edge_pass_1(y1, ei)
    y2, z, dinv = _tc_b(part1, xw1r, W2_l, W2_r, b2.reshape(1, OUT_DIM))
    part2 = _edge_pass_2(y2, ei)
    return _tc_c(part2, dinv, z)


# speedup vs baseline: 1.0268x; 1.0268x over previous
import functools

import jax
import jax.numpy as jnp
from jax import lax
from jax.experimental import pallas as pl
from jax.experimental.pallas import tpu as pltpu
from jax.experimental.pallas import tpu_sc as plsc

N = 10000
E = 320000
IN_DIM = 128
HID_DIM = 64
OUT_DIM = 8

NC = 2
NS = 16
NW = NC * NS
CHUNK = 128
NCHUNK = 80
EPW = NCHUNK * CHUNK
E_PAD = NW * EPW
NBUF = 5
N_PAD = 10240
ROWS_PER_TILE = N_PAD // NS
ZCHUNK = 128
PAD_ROW = N_PAD - 1

D1 = 72
D2 = 16


def _edge_pass(dp: int):
    mesh = plsc.VectorSubcoreMesh(core_axis_name="c", subcore_axis_name="s")

    @functools.partial(
        pl.kernel,
        out_type=jax.ShapeDtypeStruct((NC, N_PAD, 128), jnp.float32),
        mesh=mesh,
        scratch_types=[
            pltpu.VMEM((NCHUNK, CHUNK), jnp.int32),
            pltpu.VMEM((NCHUNK, CHUNK), jnp.int32),
            pltpu.VMEM((NBUF, CHUNK, dp), jnp.float32),
            pltpu.VMEM((ZCHUNK, dp), jnp.float32),
            pltpu.VMEM_SHARED((N_PAD, dp), jnp.float32),
        ] + [pltpu.SemaphoreType.DMA] * NBUF,
        compiler_params=pltpu.CompilerParams(use_tc_tiling_on_sc=False),
    )
    def k(y_hbm, ei_hbm, part_hbm, idx_s, idx_d, rows, zbuf, acc,
          *sems):
        c = lax.axis_index("c")
        s = lax.axis_index("s")
        wid = c * NS + s

        pltpu.sync_copy(ei_hbm.at[0, wid], idx_s)
        pltpu.sync_copy(ei_hbm.at[1, wid], idx_d)

        zoffs = list(range(0, dp - 15, 16))
        if zoffs[-1] + 16 < dp:
            zoffs.append(dp - 16)

        @pl.loop(0, ZCHUNK)
        def _zrows(i):
            for j in zoffs:
                zbuf[i, pl.ds(j, 16)] = jnp.zeros((16,), jnp.float32)

        @pl.loop(0, ROWS_PER_TILE // ZCHUNK)
        def _zacc(kk):
            pltpu.sync_copy(zbuf, acc.at[pl.ds(s * ROWS_PER_TILE + kk * ZCHUNK, ZCHUNK)])

        plsc.subcore_barrier()

        for b in range(NBUF):
            pltpu.async_copy(y_hbm.at[idx_s.at[b]], rows.at[b], sems[b])

        @pl.loop(0, NCHUNK // NBUF)
        def _p(p):
            for b in range(NBUF):
                g = p * NBUF + b
                pltpu.make_async_copy(y_hbm.at[idx_s.at[g]], rows.at[b],
                                      sems[b]).wait()
                pltpu.sync_copy(rows.at[b], acc.at[idx_d.at[g]], add=True)

                @pl.when(p < NCHUNK // NBUF - 1)
                def _refill():
                    pltpu.async_copy(y_hbm.at[idx_s.at[g + NBUF]], rows.at[b],
                                     sems[b])

        plsc.subcore_barrier()
        pltpu.sync_copy(
            acc.at[pl.ds(s * ROWS_PER_TILE, ROWS_PER_TILE)],
            part_hbm.at[c, pl.ds(s * ROWS_PER_TILE, ROWS_PER_TILE),
                        pl.ds(0, dp)],
        )

    return k


_edge_pass_1 = _edge_pass(D1)
_edge_pass_2 = _edge_pass(D2)


E_ROWS = E // CHUNK
PAD_ROWS = (E_PAD - E) // CHUNK


def _tc_a_body(x_ref, wl_ref, wr_ref, b1_ref, ei_ref,
               y1_ref, xw1r_ref, ei_out_ref):
    xv = x_ref[...]
    y = jnp.dot(xv, wl_ref[...], preferred_element_type=jnp.float32)
    col = lax.broadcasted_iota(jnp.int32, (N, D1 - HID_DIM), 1)
    pad = jnp.where(col == 0, 1.0, 0.0).astype(jnp.float32)
    y1_ref[...] = jnp.concatenate([y, pad], axis=1)
    xw1r_ref[...] = (jnp.dot(xv, wr_ref[...],
                             preferred_element_type=jnp.float32) + b1_ref[...])
    k = (lax.broadcasted_iota(jnp.int32, (PAD_ROWS, CHUNK), 0) * CHUNK
         + lax.broadcasted_iota(jnp.int32, (PAD_ROWS, CHUNK), 1))
    pad2 = jnp.stack([k % N, N + k % (N_PAD - N)])
    ei_out_ref[...] = jnp.concatenate(
        [ei_ref[...], pad2], axis=1).reshape(2, NW, NCHUNK, CHUNK)


_tc_a = pl.pallas_call(
    _tc_a_body,
    out_shape=(
        jax.ShapeDtypeStruct((N, D1), jnp.float32),
        jax.ShapeDtypeStruct((N, HID_DIM), jnp.float32),
        jax.ShapeDtypeStruct((2, NW, NCHUNK, CHUNK), jnp.int32),
    ),
)


def _tc_b_body(part_ref, xw1r_ref, w2l_ref, w2r_ref, b2_ref,
               y2_ref, z_ref, dinv_ref):
    p = part_ref[0, :N] + part_ref[1, :N]
    deg = p[:, HID_DIM:HID_DIM + 1]
    dinv = 1.0 / jnp.maximum(deg, 1.0)
    h = jnp.maximum(p[:, :HID_DIM] * dinv + xw1r_ref[...], 0.0)
    y2 = jnp.dot(h, w2l_ref[...], preferred_element_type=jnp.float32)
    y2_ref[...] = jnp.concatenate(
        [y2, jnp.zeros((N, D2 - OUT_DIM), jnp.float32)], axis=1)
    z_ref[...] = (jnp.dot(h, w2r_ref[...],
                          preferred_element_type=jnp.float32) + b2_ref[...])
    dinv_ref[...] = jnp.broadcast_to(dinv, (N, OUT_DIM))


_tc_b = pl.pallas_call(
    _tc_b_body,
    out_shape=(
        jax.ShapeDtypeStruct((N, D2), jnp.float32),
        jax.ShapeDtypeStruct((N, OUT_DIM), jnp.float32),
        jax.ShapeDtypeStruct((N, OUT_DIM), jnp.float32),
    ),
)


def _tc_c_body(part_ref, dinv_ref, z_ref, o_ref):
    p = part_ref[0, :N] + part_ref[1, :N]
    o_ref[...] = p[:, :OUT_DIM] * dinv_ref[...] + z_ref[...]


_tc_c = pl.pallas_call(
    _tc_c_body,
    out_shape=jax.ShapeDtypeStruct((N, OUT_DIM), jnp.float32),
)


def kernel(x, edge_index, W1_l, W1_r, b1, W2_l, W2_r, b2):
    y1, xw1r, ei = _tc_a(x, W1_l, W1_r, b1.reshape(1, HID_DIM),
                         edge_index.reshape(2, E_ROWS, CHUNK))
    part1 = _edge_pass_1(y1, ei)
    y2, z, dinv = _tc_b(part1, xw1r, W2_l, W2_r, b2.reshape(1, OUT_DIM))
    part2 = _edge_pass_2(y2, ei)
    return _tc_c(part2, dinv, z)


# speedup vs baseline: 1.0519x; 1.0244x over previous
import functools

import jax
import jax.numpy as jnp
from jax import lax
from jax.experimental import pallas as pl
from jax.experimental.pallas import tpu as pltpu
from jax.experimental.pallas import tpu_sc as plsc

N = 10000
E = 320000
IN_DIM = 128
HID_DIM = 64
OUT_DIM = 8

NC = 2
NS = 16
NW = NC * NS
CHUNK = 128
NCHUNK = 80
EPW = NCHUNK * CHUNK
E_PAD = NW * EPW
NBUF = 5
N_PAD = 10240
ROWS_PER_TILE = N_PAD // NS
ZCHUNK = 128
PAD_ROW = N_PAD - 1

D1 = 72
D2 = 16


def _edge_pass(dp: int, nbuf: int = NBUF):
    mesh = plsc.VectorSubcoreMesh(core_axis_name="c", subcore_axis_name="s")

    @functools.partial(
        pl.kernel,
        out_type=jax.ShapeDtypeStruct((NC, N_PAD, 128), jnp.float32),
        mesh=mesh,
        scratch_types=[
            pltpu.VMEM((NCHUNK, CHUNK), jnp.int32),
            pltpu.VMEM((NCHUNK, CHUNK), jnp.int32),
            pltpu.VMEM((nbuf, CHUNK, dp), jnp.float32),
            pltpu.VMEM((ZCHUNK, dp), jnp.float32),
            pltpu.VMEM_SHARED((N_PAD, dp), jnp.float32),
        ] + [pltpu.SemaphoreType.DMA] * nbuf,
        compiler_params=pltpu.CompilerParams(use_tc_tiling_on_sc=False),
    )
    def k(y_hbm, ei_hbm, part_hbm, idx_s, idx_d, rows, zbuf, acc,
          *sems):
        c = lax.axis_index("c")
        s = lax.axis_index("s")
        wid = c * NS + s

        pltpu.sync_copy(ei_hbm.at[0, wid], idx_s)
        pltpu.sync_copy(ei_hbm.at[1, wid], idx_d)

        zoffs = list(range(0, dp - 15, 16))
        if zoffs[-1] + 16 < dp:
            zoffs.append(dp - 16)

        @pl.loop(0, ZCHUNK)
        def _zrows(i):
            for j in zoffs:
                zbuf[i, pl.ds(j, 16)] = jnp.zeros((16,), jnp.float32)

        @pl.loop(0, ROWS_PER_TILE // ZCHUNK)
        def _zacc(kk):
            pltpu.sync_copy(zbuf, acc.at[pl.ds(s * ROWS_PER_TILE + kk * ZCHUNK, ZCHUNK)])

        plsc.subcore_barrier()

        for b in range(nbuf):
            pltpu.async_copy(y_hbm.at[idx_s.at[b]], rows.at[b], sems[b])

        @pl.loop(0, NCHUNK // nbuf)
        def _p(p):
            for b in range(nbuf):
                g = p * nbuf + b
                pltpu.make_async_copy(y_hbm.at[idx_s.at[g]], rows.at[b],
                                      sems[b]).wait()
                pltpu.sync_copy(rows.at[b], acc.at[idx_d.at[g]], add=True)

                @pl.when(p < NCHUNK // nbuf - 1)
                def _refill():
                    pltpu.async_copy(y_hbm.at[idx_s.at[g + nbuf]], rows.at[b],
                                     sems[b])

        plsc.subcore_barrier()
        pltpu.sync_copy(
            acc.at[pl.ds(s * ROWS_PER_TILE, ROWS_PER_TILE)],
            part_hbm.at[c, pl.ds(s * ROWS_PER_TILE, ROWS_PER_TILE),
                        pl.ds(0, dp)],
        )

    return k


_edge_pass_1 = _edge_pass(D1)
_edge_pass_2 = _edge_pass(D2, nbuf=8)


E_ROWS = E // CHUNK
PAD_ROWS = (E_PAD - E) // CHUNK


def _tc_a_body(x_ref, wl_ref, wr_ref, b1_ref, ei_ref,
               y1_ref, xw1r_ref, ei_out_ref):
    xv = x_ref[...]
    y = jnp.dot(xv, wl_ref[...], preferred_element_type=jnp.float32)
    col = lax.broadcasted_iota(jnp.int32, (N, D1 - HID_DIM), 1)
    pad = jnp.where(col == 0, 1.0, 0.0).astype(jnp.float32)
    y1_ref[...] = jnp.concatenate([y, pad], axis=1)
    xw1r_ref[...] = (jnp.dot(xv, wr_ref[...],
                             preferred_element_type=jnp.float32) + b1_ref[...])
    k = (lax.broadcasted_iota(jnp.int32, (PAD_ROWS, CHUNK), 0) * CHUNK
         + lax.broadcasted_iota(jnp.int32, (PAD_ROWS, CHUNK), 1))
    pad2 = jnp.stack([k % N, N + k % (N_PAD - N)])
    ei_out_ref[...] = jnp.concatenate(
        [ei_ref[...], pad2], axis=1).reshape(2, NW, NCHUNK, CHUNK)


_tc_a = pl.pallas_call(
    _tc_a_body,
    out_shape=(
        jax.ShapeDtypeStruct((N, D1), jnp.float32),
        jax.ShapeDtypeStruct((N, HID_DIM), jnp.float32),
        jax.ShapeDtypeStruct((2, NW, NCHUNK, CHUNK), jnp.int32),
    ),
)


def _tc_b_body(part_ref, xw1r_ref, w2l_ref, w2r_ref, b2_ref,
               y2_ref, z_ref, dinv_ref):
    p = part_ref[0, :N] + part_ref[1, :N]
    deg = p[:, HID_DIM:HID_DIM + 1]
    dinv = 1.0 / jnp.maximum(deg, 1.0)
    h = jnp.maximum(p[:, :HID_DIM] * dinv + xw1r_ref[...], 0.0)
    y2 = jnp.dot(h, w2l_ref[...], preferred_element_type=jnp.float32)
    y2_ref[...] = jnp.concatenate(
        [y2, jnp.zeros((N, D2 - OUT_DIM), jnp.float32)], axis=1)
    z_ref[...] = (jnp.dot(h, w2r_ref[...],
                          preferred_element_type=jnp.float32) + b2_ref[...])
    dinv_ref[...] = jnp.broadcast_to(dinv, (N, OUT_DIM))


_tc_b = pl.pallas_call(
    _tc_b_body,
    out_shape=(
        jax.ShapeDtypeStruct((N, D2), jnp.float32),
        jax.ShapeDtypeStruct((N, OUT_DIM), jnp.float32),
        jax.ShapeDtypeStruct((N, OUT_DIM), jnp.float32),
    ),
)


def _tc_c_body(part_ref, dinv_ref, z_ref, o_ref):
    p = part_ref[0, :N] + part_ref[1, :N]
    o_ref[...] = p[:, :OUT_DIM] * dinv_ref[...] + z_ref[...]


_tc_c = pl.pallas_call(
    _tc_c_body,
    out_shape=jax.ShapeDtypeStruct((N, OUT_DIM), jnp.float32),
)


def kernel(x, edge_index, W1_l, W1_r, b1, W2_l, W2_r, b2):
    y1, xw1r, ei = _tc_a(x, W1_l, W1_r, b1.reshape(1, HID_DIM),
                         edge_index.reshape(2, E_ROWS, CHUNK))
    part1 = _edge_pass_1(y1, ei)
    y2, z, dinv = _tc_b(part1, xw1r, W2_l, W2_r, b2.reshape(1, OUT_DIM))
    part2 = _edge_pass_2(y2, ei)
    return _tc_c(part2, dinv, z)
